# Initial kernel scaffold; baseline (speedup 1.0000x reference)
import jax
import jax.numpy as jnp
from jax.experimental import pallas as pl


def kernel(points, params):
    raise NotImplementedError("write your pallas kernel here")


# speedup vs baseline: 1.5993x; 1.5993x over previous
import functools

import jax
import jax.numpy as jnp
import numpy as np
from jax.experimental import pallas as pl
from jax.experimental.pallas import tpu as pltpu

NUM_CLASS = 18
NUM_HEADING_BIN = 12
NUM_PROPOSAL = 256
MEAN_SIZE = np.full((1, 1, NUM_CLASS, 3), 0.5, np.float32)
BN_SCALE = float(1.0 / np.sqrt(1.0 + 1e-5))


def _fps_body(x_ref, y_ref, z_ref, out_ref, dist_ref, S):
    R = x_ref.shape[1]
    x = x_ref[0]
    y = y_ref[0]
    z = z_ref[0]
    rows = jax.lax.broadcasted_iota(jnp.int32, (R, 128), 0)
    cols = jax.lax.broadcasted_iota(jnp.int32, (R, 128), 1)
    flat = rows * 128 + cols
    out_cols = jax.lax.broadcasted_iota(jnp.int32, (1, S), 1)

    dist_ref[...] = jnp.full((R, 128), 1e10, jnp.float32)
    out_ref[0] = jnp.zeros((1, S), jnp.int32)

    def body(i, last):
        onehot = (flat == last).astype(jnp.float32)
        px = jnp.sum(x * onehot)
        py = jnp.sum(y * onehot)
        pz = jnp.sum(z * onehot)
        dx = x - px
        dy = y - py
        dz = z - pz
        d = (dx * dx + dy * dy) + dz * dz
        dist = jnp.minimum(dist_ref[...], d)
        dist_ref[...] = dist
        m = jnp.max(dist)
        cand = jnp.where(dist == m, flat, jnp.int32(2**30))
        nxt = jnp.min(cand)
        out_ref[0] = out_ref[0] + jnp.where(out_cols == i, nxt, 0)
        return nxt

    jax.lax.fori_loop(1, S, body, jnp.int32(0))


def _fps_pallas(points, S, interpret=False):
    B, N, _ = points.shape
    assert N % 128 == 0
    R = N // 128
    planes = jnp.transpose(points, (2, 0, 1)).reshape(3, B, R, 128)
    x, y, z = planes[0], planes[1], planes[2]
    out = pl.pallas_call(
        functools.partial(_fps_body, S=S),
        grid=(B,),
        in_specs=[
            pl.BlockSpec((1, R, 128), lambda b: (b, 0, 0)),
            pl.BlockSpec((1, R, 128), lambda b: (b, 0, 0)),
            pl.BlockSpec((1, R, 128), lambda b: (b, 0, 0)),
        ],
        out_specs=pl.BlockSpec((1, 1, S), lambda b: (b, 0, 0)),
        out_shape=jax.ShapeDtypeStruct((B, 1, S), jnp.int32),
        scratch_shapes=[pltpu.VMEM((R, 128), jnp.float32)],
        interpret=interpret,
    )(x, y, z)
    return out.reshape(B, S)


def _batched_gather(x, idx):
    return jax.vmap(lambda xb, ib: xb[ib])(x, idx)


def _ball_grouping(points, refs, K, radius):
    d = (jnp.sum(refs ** 2, -1)[:, :, None] + jnp.sum(points ** 2, -1)[:, None, :]
         - 2.0 * jnp.einsum('bsc,bnc->bsn', refs, points))
    d = jnp.where(d <= radius * radius, d, 1e10)
    negd, idx = jax.lax.top_k(-d, K)
    valid = (-negd) < 1e9
    idx = jnp.where(valid, idx, idx[..., :1])
    return idx


def _mlp_bn(x, Ws):
    for W in Ws:
        x = jnp.maximum(jnp.einsum('...i,io->...o', x, W) * BN_SCALE, 0.0)
    return x


def _interpolate(unknown, known, known_feats, k=3):
    d = jnp.sum((unknown[:, :, None, :] - known[:, None, :, :]) ** 2, axis=-1)
    negd, idx = jax.lax.top_k(-d, k)
    d3 = jnp.maximum(-negd, 0.0)
    w = 1.0 / (d3 + 1e-8)
    w = w / jnp.sum(w, -1, keepdims=True)
    f = _batched_gather(known_feats, idx)
    return jnp.sum(f * w[..., None], axis=2)


def _aggregate(points, features, refs, Ws, K, radius, num_samples=None):
    if refs is None:
        idx = _fps_pallas(jax.lax.stop_gradient(points), num_samples)
        refs = _batched_gather(points, idx)
    nn_idx = _ball_grouping(jax.lax.stop_gradient(points), jax.lax.stop_gradient(refs), K, radius)
    nn_pts = _batched_gather(points, nn_idx)
    nn_pts = (nn_pts - refs[:, :, None, :]) / radius
    if features is not None:
        nn_f = _batched_gather(features, nn_idx)
        nn_f = jnp.concatenate([nn_pts, nn_f], axis=3)
    else:
        nn_f = nn_pts
    nn_f = _mlp_bn(nn_f, Ws)
    ref_f = jnp.max(nn_f, axis=2)
    return refs, ref_f


def kernel(points, params):
    s1_idx = _fps_pallas(points, 2048)
    s1_pts = _batched_gather(points, s1_idx)
    _, s1_fts = _aggregate(points, None, s1_pts, params['ds1'], 64, 0.2)
    s2_pts = s1_pts[:, :1024]
    _, s2_fts = _aggregate(s1_pts, s1_fts, s2_pts, params['ds2'], 32, 0.4)
    s3_pts = s1_pts[:, :512]
    _, s3_fts = _aggregate(s2_pts, s2_fts, s3_pts, params['ds3'], 16, 0.8)
    s4_pts = s1_pts[:, :256]
    _, s4_fts = _aggregate(s3_pts, s3_fts, s4_pts, params['ds4'], 16, 1.2)
    s3_fts = _mlp_bn(jnp.concatenate([s3_fts, _interpolate(s3_pts, s4_pts, s4_fts)], axis=2), params['us1'])
    s2_fts = _mlp_bn(jnp.concatenate([s2_fts, _interpolate(s2_pts, s3_pts, s3_fts)], axis=2), params['us2'])
    v = _mlp_bn(s2_fts, params['vote_mlp'])
    v = jnp.einsum('...i,io->...o', v, params['vote_final_w']) + params['vote_final_b']
    vote_xyz = s2_pts + v[..., :3]
    vote_fts = s2_fts + v[..., 3:]
    agg_pts, agg_fts = _aggregate(vote_xyz, vote_fts, None, params['prop_agg'], 16, 0.3, NUM_PROPOSAL)
    h = _mlp_bn(agg_fts, params['prop_mlp'])
    logits = jnp.einsum('...i,io->...o', h, params['prop_final_w']) + params['prop_final_b']
    B, P = logits.shape[0], logits.shape[1]
    objectness = logits[..., 0:2]
    center = agg_pts + logits[..., 2:5]
    heading_scores = logits[..., 5:17]
    heading_res = logits[..., 17:29] * (np.pi / NUM_HEADING_BIN)
    size_scores = logits[..., 29:47]
    srn = logits[..., 47:101].reshape(B, P, NUM_CLASS, 3)
    size_res = (srn * MEAN_SIZE).reshape(B, P, NUM_CLASS * 3)
    sem = logits[..., 101:119]
    return jnp.concatenate([objectness, center, heading_scores, heading_res, size_scores, size_res, sem], axis=-1)
